# Initial kernel scaffold; baseline (speedup 1.0000x reference)
import functools

import jax
import jax.numpy as jnp
from jax import lax
from jax.experimental import pallas as pl
from jax.experimental.pallas import tpu as pltpu
from jax.experimental.pallas import tpu_sc as plsc

N = 10000
E = 320000
F_IN = 128
H = 128
C = 40
H2P = 48

NC = 2
NS = 16
NW = NC * NS
LANES = 16

E_PER_TILE = E // NW
K = 80
CHUNKS = E_PER_TILE // K
N_PER_TILE = N // NS
ZR = 125

_mesh = plsc.VectorSubcoreMesh(core_axis_name="c", subcore_axis_name="s")


def _sc_degree(dst):
    @functools.partial(
        pl.kernel,
        out_type=jax.ShapeDtypeStruct((NC, N, LANES), jnp.float32),
        mesh=_mesh,
        scratch_types=[
            pltpu.VMEM((K,), jnp.int32),
            pltpu.VMEM((K, LANES), jnp.float32),
            pltpu.VMEM((ZR, LANES), jnp.float32),
            pltpu.VMEM_SHARED((N, LANES), jnp.float32),
        ],
    )
    def k(dst_h, out_h, dstv, ones, zbuf, acc):
        c = lax.axis_index("c")
        s = lax.axis_index("s")
        wid = c * NS + s

        @pl.loop(0, K)
        def _(i):
            ones[i, :] = jnp.full((LANES,), 1.0, jnp.float32)

        @pl.loop(0, ZR)
        def _(i):
            zbuf[i, :] = jnp.zeros((LANES,), jnp.float32)

        base_row = s * N_PER_TILE

        @pl.loop(0, N_PER_TILE // ZR)
        def _(r):
            pltpu.sync_copy(zbuf, acc.at[pl.ds(base_row + r * ZR, ZR)])

        plsc.subcore_barrier()

        ebase = wid * E_PER_TILE

        @pl.loop(0, CHUNKS)
        def _(ci):
            pltpu.sync_copy(dst_h.at[pl.ds(ebase + ci * K, K)], dstv)
            pltpu.sync_copy(ones, acc.at[dstv], add=True)

        plsc.subcore_barrier()
        pltpu.sync_copy(
            acc.at[pl.ds(base_row, N_PER_TILE)],
            out_h.at[c, pl.ds(base_row, N_PER_TILE)],
        )

    return k(dst)


def _sc_scatter(table, src, dst, d):
    @functools.partial(
        pl.kernel,
        out_type=jax.ShapeDtypeStruct((NC, N, d), jnp.float32),
        mesh=_mesh,
        scratch_types=[
            pltpu.VMEM((K,), jnp.int32),
            pltpu.VMEM((K,), jnp.int32),
            pltpu.VMEM((K, d), jnp.float32),
            pltpu.VMEM((ZR, d), jnp.float32),
            pltpu.VMEM_SHARED((N, d), jnp.float32),
            pltpu.SemaphoreType.DMA,
        ],
    )
    def k(table_h, src_h, dst_h, out_h, srcv, dstv, rows, zbuf, acc, sem):
        c = lax.axis_index("c")
        s = lax.axis_index("s")
        wid = c * NS + s

        @pl.loop(0, ZR)
        def _(i):
            @pl.loop(0, d, step=LANES)
            def _(j):
                zbuf[i, pl.ds(j, LANES)] = jnp.zeros((LANES,), jnp.float32)

        base_row = s * N_PER_TILE

        @pl.loop(0, N_PER_TILE // ZR)
        def _(r):
            pltpu.sync_copy(zbuf, acc.at[pl.ds(base_row + r * ZR, ZR)])

        plsc.subcore_barrier()

        ebase = wid * E_PER_TILE

        @pl.loop(0, CHUNKS)
        def _(ci):
            pltpu.sync_copy(src_h.at[pl.ds(ebase + ci * K, K)], srcv)
            pltpu.sync_copy(dst_h.at[pl.ds(ebase + ci * K, K)], dstv)
            pltpu.async_copy(table_h.at[srcv], rows, sem).wait()
            pltpu.sync_copy(rows, acc.at[dstv], add=True)

        plsc.subcore_barrier()
        pltpu.sync_copy(
            acc.at[pl.ds(base_row, N_PER_TILE)],
            out_h.at[c, pl.ds(base_row, N_PER_TILE)],
        )

    return k(table, src, dst)


RB = 1000


def _dinv_of(degp0, degp1):
    deg = degp0 + degp1 + 1.0
    return lax.rsqrt(deg)[:, 0:1]


def _stage1_body(x_ref, w1_ref, degp_ref, hs1_ref):
    dinv = _dinv_of(degp_ref[0], degp_ref[1])
    h = jnp.dot(x_ref[...], w1_ref[...], preferred_element_type=jnp.float32)
    hs1_ref[...] = h * dinv


def _stage2_body(s1p_ref, hs1_ref, degp_ref, w2_ref, b1_ref, h1_ref, hs2_ref):
    dinv = _dinv_of(degp_ref[0], degp_ref[1])
    s1 = s1p_ref[0] + s1p_ref[1] + hs1_ref[...]
    h1 = s1 * dinv + b1_ref[...]
    h1_ref[...] = h1
    h1a = jnp.maximum(h1, 0.0)
    hs2_ref[...] = (
        jnp.dot(h1a, w2_ref[...], preferred_element_type=jnp.float32) * dinv
    )


def _stage3_body(
    s2p_ref, hs2_ref, degp_ref, h1_ref, wa_ref, wb_ref, b2_ref, blin_ref, out_ref
):
    dinv = _dinv_of(degp_ref[0], degp_ref[1])
    h2 = (s2p_ref[0] + s2p_ref[1] + hs2_ref[...]) * dinv + b2_ref[...]
    final = (
        jnp.dot(h1_ref[...], wa_ref[...], preferred_element_type=jnp.float32)
        + jnp.dot(h2, wb_ref[...], preferred_element_type=jnp.float32)
        + blin_ref[...]
    )
    m = jnp.max(final, axis=1, keepdims=True)
    e = jnp.exp(final - m)
    lse = jnp.log(jnp.sum(e, axis=1, keepdims=True))
    out_ref[...] = final - m - lse


def _row_spec(d):
    return pl.BlockSpec((RB, d), lambda i: (i, 0))


def _part_spec(d):
    return pl.BlockSpec((2, RB, d), lambda i: (0, i, 0))


def _full_spec(r, c_):
    return pl.BlockSpec((r, c_), lambda i: (0, 0))


def kernel(x, edge_index, W1, b1, W2, b2, Wlin, blin):
    src = edge_index[0]
    dst = edge_index[1]

    degp = _sc_degree(dst)

    hs1 = pl.pallas_call(
        _stage1_body,
        grid=(N // RB,),
        in_specs=[_row_spec(F_IN), _full_spec(F_IN, H), _part_spec(LANES)],
        out_specs=_row_spec(H),
        out_shape=jax.ShapeDtypeStruct((N, H), jnp.float32),
    )(x, W1, degp)

    s1p = _sc_scatter(hs1, src, dst, H)

    W2p = jnp.pad(W2, ((0, 0), (0, H2P - C)))
    b1r = b1.reshape(1, H)
    h1, hs2 = pl.pallas_call(
        _stage2_body,
        grid=(N // RB,),
        in_specs=[
            _part_spec(H),
            _row_spec(H),
            _part_spec(LANES),
            _full_spec(H, H2P),
            _full_spec(1, H),
        ],
        out_specs=[_row_spec(H), _row_spec(H2P)],
        out_shape=[
            jax.ShapeDtypeStruct((N, H), jnp.float32),
            jax.ShapeDtypeStruct((N, H2P), jnp.float32),
        ],
    )(s1p, hs1, degp, W2p, b1r)

    s2p = _sc_scatter(hs2, src, dst, H2P)

    Wa = Wlin[:H]
    Wb = jnp.pad(Wlin[H:], ((0, H2P - C), (0, 0)))
    b2r = jnp.pad(b2, (0, H2P - C)).reshape(1, H2P)
    blinr = blin.reshape(1, C)
    out = pl.pallas_call(
        _stage3_body,
        grid=(N // RB,),
        in_specs=[
            _part_spec(H2P),
            _row_spec(H2P),
            _part_spec(LANES),
            _row_spec(H),
            _full_spec(H, C),
            _full_spec(H2P, C),
            _full_spec(1, H2P),
            _full_spec(1, C),
        ],
        out_specs=_row_spec(C),
        out_shape=jax.ShapeDtypeStruct((N, C), jnp.float32),
    )(s2p, hs2, degp, h1, Wa, Wb, b2r, blinr)
    return out


# speedup vs baseline: 12.1914x; 12.1914x over previous
import functools

import jax
import jax.numpy as jnp
from jax import lax
from jax.experimental import pallas as pl
from jax.experimental.pallas import tpu as pltpu
from jax.experimental.pallas import tpu_sc as plsc

N = 10000
E = 320000
F_IN = 128
H = 128
C = 40

NC = 2
NS = 16
NW = NC * NS
LANES = 16

E_PER_TILE = E // NW
K = 80
CHUNKS = E_PER_TILE // K
N_PAD = 10240
N_PER_TILE = N_PAD // NS
ZR = 128

_mesh = plsc.VectorSubcoreMesh(core_axis_name="c", subcore_axis_name="s")


def _sc_degree(dst):
    @functools.partial(
        pl.kernel,
        out_type=jax.ShapeDtypeStruct((NC, N_PAD, H), jnp.float32),
        mesh=_mesh,
        scratch_types=[
            pltpu.VMEM((K,), jnp.int32),
            pltpu.VMEM((K, H), jnp.float32),
            pltpu.VMEM((ZR, H), jnp.float32),
            pltpu.VMEM_SHARED((N_PAD, H), jnp.float32),
        ],
    )
    def k(dst_h, out_h, dstv, ones, zbuf, acc):
        c = lax.axis_index("c")
        s = lax.axis_index("s")
        wid = c * NS + s

        @pl.loop(0, K)
        def _(i):
            @pl.loop(0, H, step=LANES)
            def _(j):
                ones[i, pl.ds(j, LANES)] = jnp.full((LANES,), 1.0, jnp.float32)

        @pl.loop(0, ZR)
        def _(i):
            @pl.loop(0, H, step=LANES)
            def _(j):
                zbuf[i, pl.ds(j, LANES)] = jnp.zeros((LANES,), jnp.float32)

        base_row = s * N_PER_TILE

        @pl.loop(0, N_PER_TILE // ZR)
        def _(r):
            pltpu.sync_copy(zbuf, acc.at[pl.ds(base_row + r * ZR, ZR)])

        plsc.subcore_barrier()

        ebase = wid * E_PER_TILE

        @pl.loop(0, CHUNKS)
        def _(ci):
            pltpu.sync_copy(dst_h.at[pl.ds(ebase + ci * K, K)], dstv)
            pltpu.sync_copy(ones, acc.at[dstv], add=True)

        plsc.subcore_barrier()
        pltpu.sync_copy(
            acc.at[pl.ds(base_row, N_PER_TILE)],
            out_h.at[c, pl.ds(base_row, N_PER_TILE)],
        )

    return k(dst)


def _sc_scatter(table, src, dst, d):
    @functools.partial(
        pl.kernel,
        out_type=jax.ShapeDtypeStruct((NC, N_PAD, d), jnp.float32),
        mesh=_mesh,
        scratch_types=[
            pltpu.VMEM((K,), jnp.int32),
            pltpu.VMEM((K,), jnp.int32),
            pltpu.VMEM((K, d), jnp.float32),
            pltpu.VMEM((ZR, d), jnp.float32),
            pltpu.VMEM_SHARED((N_PAD, d), jnp.float32),
            pltpu.SemaphoreType.DMA,
        ],
    )
    def k(table_h, src_h, dst_h, out_h, srcv, dstv, rows, zbuf, acc, sem):
        c = lax.axis_index("c")
        s = lax.axis_index("s")
        wid = c * NS + s

        @pl.loop(0, ZR)
        def _(i):
            @pl.loop(0, d, step=LANES)
            def _(j):
                zbuf[i, pl.ds(j, LANES)] = jnp.zeros((LANES,), jnp.float32)

        base_row = s * N_PER_TILE

        @pl.loop(0, N_PER_TILE // ZR)
        def _(r):
            pltpu.sync_copy(zbuf, acc.at[pl.ds(base_row + r * ZR, ZR)])

        plsc.subcore_barrier()

        ebase = wid * E_PER_TILE

        @pl.loop(0, CHUNKS)
        def _(ci):
            pltpu.sync_copy(src_h.at[pl.ds(ebase + ci * K, K)], srcv)
            pltpu.sync_copy(dst_h.at[pl.ds(ebase + ci * K, K)], dstv)
            pltpu.async_copy(table_h.at[srcv], rows, sem).wait()
            pltpu.sync_copy(rows, acc.at[dstv], add=True)

        plsc.subcore_barrier()
        pltpu.sync_copy(
            acc.at[pl.ds(base_row, N_PER_TILE)],
            out_h.at[c, pl.ds(base_row, N_PER_TILE)],
        )

    return k(table, src, dst)


RB = 1024


def _dinv_of(degp0, degp1):
    deg = degp0 + degp1 + 1.0
    return lax.rsqrt(deg)[:, 0:1]


def _stage1_body(x_ref, w1_ref, degp_ref, hs1_ref):
    dinv = _dinv_of(degp_ref[0], degp_ref[1])
    h = jnp.dot(x_ref[...], w1_ref[...], preferred_element_type=jnp.float32)
    hs1_ref[...] = h * dinv


def _stage2_body(s1p_ref, hs1_ref, degp_ref, b1_ref, h1_ref, g_ref):
    dinv = _dinv_of(degp_ref[0], degp_ref[1])
    s1 = s1p_ref[0] + s1p_ref[1] + hs1_ref[...]
    h1 = s1 * dinv + b1_ref[...]
    h1_ref[...] = h1
    g_ref[...] = jnp.maximum(h1, 0.0) * dinv


def _stage3_body(
    s2p_ref, g_ref, degp_ref, h1_ref, w2_ref, wa_ref, wb_ref, b2_ref, blin_ref,
    out_ref,
):
    dinv = _dinv_of(degp_ref[0], degp_ref[1])
    t = (s2p_ref[0] + s2p_ref[1] + g_ref[...]) * dinv
    h2 = jnp.dot(t, w2_ref[...], preferred_element_type=jnp.float32) + b2_ref[...]
    final = (
        jnp.dot(h1_ref[...], wa_ref[...], preferred_element_type=jnp.float32)
        + jnp.dot(h2, wb_ref[...], preferred_element_type=jnp.float32)
        + blin_ref[...]
    )
    m = jnp.max(final, axis=1, keepdims=True)
    e = jnp.exp(final - m)
    lse = jnp.log(jnp.sum(e, axis=1, keepdims=True))
    out_ref[...] = final - m - lse


def _row_spec(d):
    return pl.BlockSpec((RB, d), lambda i: (i, 0))


def _part_spec(d):
    return pl.BlockSpec((2, RB, d), lambda i: (0, i, 0))


def _full_spec(r, c_):
    return pl.BlockSpec((r, c_), lambda i: (0, 0))


def kernel(x, edge_index, W1, b1, W2, b2, Wlin, blin):
    src = edge_index[0]
    dst = edge_index[1]
    xp = jnp.pad(x, ((0, N_PAD - N), (0, 0)))

    degp = _sc_degree(dst)

    hs1 = pl.pallas_call(
        _stage1_body,
        grid=(N_PAD // RB,),
        in_specs=[_row_spec(F_IN), _full_spec(F_IN, H), _part_spec(H)],
        out_specs=_row_spec(H),
        out_shape=jax.ShapeDtypeStruct((N_PAD, H), jnp.float32),
    )(xp, W1, degp)

    s1p = _sc_scatter(hs1, src, dst, H)

    b1r = b1.reshape(1, H)
    h1, g = pl.pallas_call(
        _stage2_body,
        grid=(N_PAD // RB,),
        in_specs=[
            _part_spec(H),
            _row_spec(H),
            _part_spec(H),
            _full_spec(1, H),
        ],
        out_specs=[_row_spec(H), _row_spec(H)],
        out_shape=[
            jax.ShapeDtypeStruct((N_PAD, H), jnp.float32),
            jax.ShapeDtypeStruct((N_PAD, H), jnp.float32),
        ],
    )(s1p, hs1, degp, b1r)

    s2p = _sc_scatter(g, src, dst, H)

    Wa = Wlin[:H]
    Wb = Wlin[H:]
    b2r = b2.reshape(1, C)
    blinr = blin.reshape(1, C)
    out = pl.pallas_call(
        _stage3_body,
        grid=(N_PAD // RB,),
        in_specs=[
            _part_spec(H),
            _row_spec(H),
            _part_spec(H),
            _row_spec(H),
            _full_spec(H, C),
            _full_spec(H, C),
            _full_spec(C, C),
            _full_spec(1, C),
            _full_spec(1, C),
        ],
        out_specs=_row_spec(C),
        out_shape=jax.ShapeDtypeStruct((N_PAD, C), jnp.float32),
    )(s2p, g, degp, h1, W2, Wa, Wb, b2r, blinr)
    return out[:N]


# speedup vs baseline: 32.2489x; 2.6452x over previous
import dataclasses
import functools

import jax
import jax.numpy as jnp
from jax import lax
from jax.experimental import pallas as pl
from jax.experimental.pallas import tpu as pltpu
from jax.experimental.pallas import tpu_sc as plsc

N = 10000
E = 320000
F_IN = 128
H = 128
C = 40

NC = 2
NS = 16
NW = NC * NS
LANES = 16

E_PER_TILE = E // NW
EPT = 10240
E_PAD = EPT * NW
N_PAD = 10112
N_PER_TILE = N_PAD // NS
ZR = 8
KD = 128
DNF = EPT // KD
DRING = 5
KS = 80
SNF = EPT // KS
CR = 4
NR = SNF // CR

_mesh = plsc.VectorSubcoreMesh(core_axis_name="c", subcore_axis_name="s")
_CP = pltpu.CompilerParams()
if "needs_layout_passes" in pltpu.CompilerParams.__dataclass_fields__:
    _CP = dataclasses.replace(_CP, needs_layout_passes=False)


def _sc_degree(dst):
    @functools.partial(
        pl.kernel,
        out_type=jax.ShapeDtypeStruct((NW, N_PAD), jnp.float32),
        mesh=_mesh,
        compiler_params=_CP,
        scratch_types=[
            pltpu.VMEM((N_PAD,), jnp.float32),
            pltpu.VMEM((EPT,), jnp.int32),
        ],
    )
    def k(dst_h, out_h, hist, dstb):
        c = lax.axis_index("c")
        s = lax.axis_index("s")
        wid = c * NS + s

        @pl.loop(0, N_PAD // LANES)
        def _(i):
            hist[pl.ds(i * LANES, LANES)] = jnp.zeros((LANES,), jnp.float32)

        pltpu.sync_copy(dst_h.at[pl.ds(wid * EPT, EPT)], dstb)
        one = jnp.full((LANES,), 1.0, jnp.float32)

        @pl.loop(0, EPT // LANES)
        def _(i):
            iv = dstb[pl.ds(i * LANES, LANES)]
            plsc.addupdate_scatter(hist, [iv], one)

        pltpu.sync_copy(hist, out_h.at[wid])

    return k(dst)


def _sc_scatter(table, eidx, d):
    @functools.partial(
        pl.kernel,
        out_type=jax.ShapeDtypeStruct((NC, N_PAD, d), jnp.float32),
        mesh=_mesh,
        scratch_types=(
            [pltpu.VMEM((2, KS), jnp.int32)] * 8
            + [pltpu.VMEM((KS, d), jnp.float32)] * 4
            + [
                pltpu.VMEM((ZR, d), jnp.float32),
                pltpu.VMEM_SHARED((N_PAD, d), jnp.float32),
            ]
            + [pltpu.SemaphoreType.DMA] * 2
            + [pltpu.SemaphoreType.DMA] * 2
            + [pltpu.SemaphoreType.DMA]
        ),
    )
    def k(table_h, eidx_h, out_h, *refs):
        idxb = refs[0:8]
        rows = refs[8:12]
        zbuf = refs[12]
        acc = refs[13]
        gsem = refs[14:16]
        asem = refs[16:18]
        isem = refs[18]
        c = lax.axis_index("c")
        s = lax.axis_index("s")
        wid = c * NS + s

        @pl.loop(0, ZR)
        def _(i):
            @pl.loop(0, d, step=LANES)
            def _(j):
                zbuf[i, pl.ds(j, LANES)] = jnp.zeros((LANES,), jnp.float32)

        base_row = s * N_PER_TILE

        @pl.loop(0, N_PER_TILE // ZR)
        def _(r):
            pltpu.sync_copy(zbuf, acc.at[pl.ds(base_row + r * ZR, ZR)])

        plsc.subcore_barrier()

        cbase = wid * SNF

        for j in range(2):
            pltpu.sync_copy(eidx_h.at[cbase + j], idxb[j])
            pltpu.async_copy(table_h.at[idxb[j].at[0]], rows[j], gsem[j])
        pltpu.async_copy(eidx_h.at[cbase + 2], idxb[2], isem)

        NIT = SNF // 8

        @pl.loop(0, NIT)
        def _(it):
            cn0 = it * 8
            for j in range(8):
                r4 = j % 4
                sg = j % 2
                pltpu.make_async_copy(
                    table_h.at[idxb[j].at[0]], rows[r4], gsem[sg]
                ).wait()

                def wait_add():
                    pltpu.make_async_copy(
                        rows[(j - 2) % 4],
                        acc.at[idxb[(j - 2) % 8].at[1]],
                        asem[sg],
                    ).wait()

                if j < 2:

                    @pl.when(it > 0)
                    def _():
                        wait_add()

                else:
                    wait_add()
                pltpu.async_copy(
                    rows[r4], acc.at[idxb[j].at[1]], asem[sg], add=True
                )

                def prefetch(fire_gather, fire_idx):
                    if fire_gather:
                        pltpu.make_async_copy(
                            eidx_h.at[cbase + cn0 + j + 2], idxb[(j + 2) % 8],
                            isem,
                        ).wait()
                        pltpu.async_copy(
                            table_h.at[idxb[(j + 2) % 8].at[0]],
                            rows[(j + 2) % 4],
                            gsem[sg],
                        )
                    if fire_idx:
                        pltpu.async_copy(
                            eidx_h.at[cbase + cn0 + j + 3], idxb[(j + 3) % 8],
                            isem,
                        )

                if j < 5:
                    prefetch(True, True)
                elif j < 6:

                    @pl.when(it + 1 < NIT)
                    def _():
                        prefetch(False, True)

                    prefetch(True, False)
                else:

                    @pl.when(it + 1 < NIT)
                    def _():
                        prefetch(True, True)

        for j in range(2):
            pltpu.make_async_copy(
                rows[2 + j], acc.at[idxb[6 + j].at[1]], asem[j]
            ).wait()

        plsc.subcore_barrier()
        pltpu.sync_copy(
            acc.at[pl.ds(base_row, N_PER_TILE)],
            out_h.at[c, pl.ds(base_row, N_PER_TILE)],
        )

    return k(table, eidx)


RB = 632


def _dinv_of(degp_ref):
    deg = jnp.sum(degp_ref[0], axis=0) + 1.0
    return lax.rsqrt(deg).reshape(RB, 1)


def _stage1_body(x_ref, w1_ref, degp_ref, hs1_ref):
    dinv = _dinv_of(degp_ref)
    h = jnp.dot(x_ref[...], w1_ref[...], preferred_element_type=jnp.float32)
    hs1_ref[...] = h * dinv


def _stage2_body(s1p_ref, hs1_ref, degp_ref, b1_ref, h1_ref, g_ref):
    dinv = _dinv_of(degp_ref)
    s1 = s1p_ref[0] + s1p_ref[1] + hs1_ref[...]
    h1 = s1 * dinv + b1_ref[...]
    h1_ref[...] = h1
    g_ref[...] = jnp.maximum(h1, 0.0) * dinv


def _stage3_body(
    s2p_ref, g_ref, degp_ref, h1_ref, w2_ref, wa_ref, wb_ref, b2_ref, blin_ref,
    out_ref,
):
    dinv = _dinv_of(degp_ref)
    t = (s2p_ref[0] + s2p_ref[1] + g_ref[...]) * dinv
    h2 = jnp.dot(t, w2_ref[...], preferred_element_type=jnp.float32) + b2_ref[...]
    final = (
        jnp.dot(h1_ref[...], wa_ref[...], preferred_element_type=jnp.float32)
        + jnp.dot(h2, wb_ref[...], preferred_element_type=jnp.float32)
        + blin_ref[...]
    )
    m = jnp.max(final, axis=1, keepdims=True)
    e = jnp.exp(final - m)
    lse = jnp.log(jnp.sum(e, axis=1, keepdims=True))
    out_ref[...] = final - m - lse


def _row_spec(d):
    return pl.BlockSpec((RB, d), lambda i: (i, 0))


def _part_spec(d):
    return pl.BlockSpec((2, RB, d), lambda i: (0, i, 0))


_deg_spec = pl.BlockSpec((1, NW, RB), lambda i: (i, 0, 0))


def _full_spec(r, c_):
    return pl.BlockSpec((r, c_), lambda i: (0, 0))


def kernel(x, edge_index, W1, b1, W2, b2, Wlin, blin):
    pad_ids = N + jnp.arange(EPT - E_PER_TILE, dtype=jnp.int32) % (N_PAD - N)
    pad_blk = jnp.broadcast_to(pad_ids, (2, NW, EPT - E_PER_TILE))
    eip = jnp.concatenate(
        [edge_index.reshape(2, NW, E_PER_TILE), pad_blk], axis=2
    ).reshape(2, E_PAD)
    src = eip[0]
    dst = eip[1]
    eic = eip.reshape(2, NW, SNF, KS).transpose(1, 2, 0, 3).reshape(NW * SNF, 2, KS)
    xp = jnp.pad(x, ((0, N_PAD - N), (0, 0)))

    degp = _sc_degree(dst).reshape(NW, NS, N_PER_TILE).transpose(1, 0, 2)

    hs1 = pl.pallas_call(
        _stage1_body,
        grid=(N_PAD // RB,),
        in_specs=[_row_spec(F_IN), _full_spec(F_IN, H), _deg_spec],
        out_specs=_row_spec(H),
        out_shape=jax.ShapeDtypeStruct((N_PAD, H), jnp.float32),
    )(xp, W1, degp)

    s1p = _sc_scatter(hs1, eic, H)

    b1r = b1.reshape(1, H)
    h1, g = pl.pallas_call(
        _stage2_body,
        grid=(N_PAD // RB,),
        in_specs=[
            _part_spec(H),
            _row_spec(H),
            _deg_spec,
            _full_spec(1, H),
        ],
        out_specs=[_row_spec(H), _row_spec(H)],
        out_shape=[
            jax.ShapeDtypeStruct((N_PAD, H), jnp.float32),
            jax.ShapeDtypeStruct((N_PAD, H), jnp.float32),
        ],
    )(s1p, hs1, degp, b1r)

    s2p = _sc_scatter(g, eic, H)

    Wa = Wlin[:H]
    Wb = Wlin[H:]
    b2r = b2.reshape(1, C)
    blinr = blin.reshape(1, C)
    out = pl.pallas_call(
        _stage3_body,
        grid=(N_PAD // RB,),
        in_specs=[
            _part_spec(H),
            _row_spec(H),
            _deg_spec,
            _row_spec(H),
            _full_spec(H, C),
            _full_spec(H, C),
            _full_spec(C, C),
            _full_spec(1, C),
            _full_spec(1, C),
        ],
        out_specs=_row_spec(C),
        out_shape=jax.ShapeDtypeStruct((N_PAD, C), jnp.float32),
    )(s2p, g, degp, h1, W2, Wa, Wb, b2r, blinr)
    return out[:N]
